# baseline (speedup 1.0000x reference)
import functools

import jax
import jax.numpy as jnp
from jax import lax
from jax.experimental import pallas as pl
from jax.experimental.pallas import tpu as pltpu
from jax.experimental.pallas import tpu_sc as plsc

_N = 256
_Q = 16384
_NC, _NS, _L = 2, 16, 16
_NW = _NC * _NS
_QPW = _Q // _NW
_ITERS = _QPW // _L
_HALF_ROWS = _N // 2
_INV_H = (_N - 1) / 2.0


def _f1(a):
    return (1.5 * a - 2.5) * a * a + 1.0


def _f2(a):
    return ((-0.5 * a + 2.5) * a - 4.0) * a + 2.0


_mesh = plsc.VectorSubcoreMesh(core_axis_name="c", subcore_axis_name="s")


@functools.partial(
    pl.kernel,
    out_type=jax.ShapeDtypeStruct((_Q,), jnp.float32),
    mesh=_mesh,
    scratch_types=[
        pltpu.VMEM((_N * _N // 2,), jnp.int32),
        pltpu.VMEM_SHARED((_N * _N // 4,), jnp.int32),
        pltpu.VMEM((_QPW,), jnp.int32),
        pltpu.VMEM((_QPW,), jnp.int32),
        pltpu.VMEM((_QPW,), jnp.float32),
        pltpu.SemaphoreType.DMA,
        pltpu.SemaphoreType.DMA,
    ],
    compiler_params=pltpu.CompilerParams(needs_layout_passes=False),
)
def _spline_sc(buf_hbm, out_hbm,
               w_v, w_sh, x0_v, x1_v, out_v, wsem, xsem):
    sid = lax.axis_index("s")
    wid = sid * _NC + lax.axis_index("c")
    base = wid * _QPW

    c0 = pltpu.make_async_copy(
        buf_hbm.at[pl.ds(_N * _N // 2 + base, _QPW)], x0_v, xsem)
    c1 = pltpu.make_async_copy(
        buf_hbm.at[pl.ds(_N * _N // 2 + _Q + base, _QPW)], x1_v, xsem)
    half = _N * _N // 4
    cw2 = pltpu.make_async_copy(buf_hbm.at[pl.ds(half, half)],
                                w_v.at[pl.ds(half, half)], wsem)
    c0.start()
    c1.start()
    cw2.start()

    @pl.when(sid == 0)
    def _():
        pltpu.sync_copy(buf_hbm.at[pl.ds(0, half)], w_sh)

    plsc.subcore_barrier()
    cw1 = pltpu.make_async_copy(w_sh, w_v.at[pl.ds(0, half)], wsem)
    cw1.start()
    c0.wait()
    c1.wait()
    cw2.wait()
    cw1.wait()

    def body(i, _):
        off = i * _L
        fx = (plsc.bitcast(x0_v[pl.ds(off, _L)], jnp.float32) + 1.0) * _INV_H
        fy = (plsc.bitcast(x1_v[pl.ds(off, _L)], jnp.float32) + 1.0) * _INV_H
        ix = jnp.minimum(fx.astype(jnp.int32), _N - 2)
        iy = jnp.minimum(fy.astype(jnp.int32), _N - 2)
        u = fx - ix.astype(jnp.float32)
        v = fy - iy.astype(jnp.float32)

        zero = jnp.zeros((_L,), jnp.float32)
        cx = (jnp.where(ix >= 1, _f2(1.0 + u), zero),
              _f1(u),
              _f1(1.0 - u),
              jnp.where(ix <= _N - 3, _f2(2.0 - u), zero))
        cy = (jnp.where(iy >= 1, _f2(1.0 + v), zero),
              _f1(v),
              _f1(1.0 - v),
              jnp.where(iy <= _N - 3, _f2(2.0 - v), zero))

        cols = (jnp.maximum(ix - 1, 0), ix, ix + 1,
                jnp.minimum(ix + 2, _N - 1))
        rows = (jnp.maximum(iy - 1, 0), iy, iy + 1,
                jnp.minimum(iy + 2, _N - 1))

        acc = zero
        for j in range(4):
            wrow = (rows[j] & (_HALF_ROWS - 1)) * _N
            hij = rows[j] >= _HALF_ROWS
            s = zero
            for k in range(4):
                g = plsc.load_gather(w_v, [wrow + cols[k]])
                lo = plsc.bitcast(g << 16, jnp.float32)
                hi = plsc.bitcast(g & jnp.int32(-65536), jnp.float32)
                s += jnp.where(hij, hi, lo) * cx[k]
            acc += s * cy[j]
        out_v[pl.ds(off, _L)] = acc
        return 0

    lax.fori_loop(0, _ITERS, body, 0)
    pltpu.sync_copy(out_v, out_hbm.at[pl.ds(base, _QPW)])


def kernel(x, weights):
    xt = x.T
    w32 = lax.bitcast_convert_type(weights.reshape(-1), jnp.int32) + 0x8000
    lo = (w32[: _N * _N // 2] >> 16) & 0xFFFF
    hi = w32[_N * _N // 2 :] & jnp.int32(-65536)
    buf = jnp.concatenate([lo | hi,
                           lax.bitcast_convert_type(xt[0], jnp.int32),
                           lax.bitcast_convert_type(xt[1], jnp.int32)])
    out = _spline_sc(buf)
    return (out, x)


# speedup vs baseline: 1.0573x; 1.0573x over previous
import functools

import jax
import jax.numpy as jnp
from jax import lax
from jax.experimental import pallas as pl
from jax.experimental.pallas import tpu as pltpu
from jax.experimental.pallas import tpu_sc as plsc

_N = 256
_Q = 16384
_NC, _NS, _L = 2, 16, 16
_NW = _NC * _NS
_QPW = _Q // _NW
_ITERS = _QPW // _L
_HALF_ROWS = _N // 2
_INV_H = (_N - 1) / 2.0


def _f1(a):
    return (1.5 * a - 2.5) * a * a + 1.0


def _f2(a):
    return ((-0.5 * a + 2.5) * a - 4.0) * a + 2.0


_mesh = plsc.VectorSubcoreMesh(core_axis_name="c", subcore_axis_name="s")


@functools.partial(
    pl.kernel,
    out_type=jax.ShapeDtypeStruct((_Q,), jnp.float32),
    mesh=_mesh,
    scratch_types=[
        pltpu.VMEM((_N * _N // 2,), jnp.int32),
        pltpu.VMEM_SHARED((_N * _N // 2,), jnp.int32),
        pltpu.VMEM((_QPW,), jnp.int32),
        pltpu.VMEM((_QPW,), jnp.int32),
        pltpu.VMEM((_QPW,), jnp.float32),
        pltpu.SemaphoreType.DMA,
        pltpu.SemaphoreType.DMA,
    ],
    compiler_params=pltpu.CompilerParams(needs_layout_passes=False),
)
def _spline_sc(buf_hbm, out_hbm,
               w_v, w_sh, x0_v, x1_v, out_v, wsem, xsem):
    sid = lax.axis_index("s")
    wid = sid * _NC + lax.axis_index("c")
    base = wid * _QPW

    c0 = pltpu.make_async_copy(
        buf_hbm.at[pl.ds(_N * _N // 2 + base, _QPW)], x0_v, xsem)
    c1 = pltpu.make_async_copy(
        buf_hbm.at[pl.ds(_N * _N // 2 + _Q + base, _QPW)], x1_v, xsem)
    c0.start()
    c1.start()

    @pl.when(sid == 0)
    def _():
        pltpu.sync_copy(buf_hbm.at[pl.ds(0, _N * _N // 2)], w_sh)

    plsc.subcore_barrier()
    cw = pltpu.make_async_copy(w_sh, w_v, wsem)
    cw.start()
    c0.wait()
    c1.wait()
    cw.wait()

    def body(i, _):
        off = i * _L
        fx = (plsc.bitcast(x0_v[pl.ds(off, _L)], jnp.float32) + 1.0) * _INV_H
        fy = (plsc.bitcast(x1_v[pl.ds(off, _L)], jnp.float32) + 1.0) * _INV_H
        ix = jnp.minimum(fx.astype(jnp.int32), _N - 2)
        iy = jnp.minimum(fy.astype(jnp.int32), _N - 2)
        u = fx - ix.astype(jnp.float32)
        v = fy - iy.astype(jnp.float32)

        zero = jnp.zeros((_L,), jnp.float32)
        cx = (jnp.where(ix >= 1, _f2(1.0 + u), zero),
              _f1(u),
              _f1(1.0 - u),
              jnp.where(ix <= _N - 3, _f2(2.0 - u), zero))
        cy = (jnp.where(iy >= 1, _f2(1.0 + v), zero),
              _f1(v),
              _f1(1.0 - v),
              jnp.where(iy <= _N - 3, _f2(2.0 - v), zero))

        cols = (jnp.maximum(ix - 1, 0), ix, ix + 1,
                jnp.minimum(ix + 2, _N - 1))
        rows = (jnp.maximum(iy - 1, 0), iy, iy + 1,
                jnp.minimum(iy + 2, _N - 1))

        acc = zero
        for j in range(4):
            wrow = (rows[j] & (_HALF_ROWS - 1)) * _N
            hij = rows[j] >= _HALF_ROWS
            s = zero
            for k in range(4):
                g = plsc.load_gather(w_v, [wrow + cols[k]])
                lo = plsc.bitcast(g << 16, jnp.float32)
                hi = plsc.bitcast(g & jnp.int32(-65536), jnp.float32)
                s += jnp.where(hij, hi, lo) * cx[k]
            acc += s * cy[j]
        out_v[pl.ds(off, _L)] = acc
        return 0

    lax.fori_loop(0, _ITERS, body, 0)
    pltpu.sync_copy(out_v, out_hbm.at[pl.ds(base, _QPW)])


def kernel(x, weights):
    xt = x.T
    w32 = lax.bitcast_convert_type(weights.reshape(-1), jnp.int32) + 0x8000
    lo = (w32[: _N * _N // 2] >> 16) & 0xFFFF
    hi = w32[_N * _N // 2 :] & jnp.int32(-65536)
    buf = jnp.concatenate([lo | hi,
                           lax.bitcast_convert_type(xt[0], jnp.int32),
                           lax.bitcast_convert_type(xt[1], jnp.int32)])
    out = _spline_sc(buf)
    return (out, x)
